# Initial kernel scaffold; baseline (speedup 1.0000x reference)
import jax
import jax.numpy as jnp
from jax.experimental import pallas as pl


def kernel(x, table):
    raise NotImplementedError("write your pallas kernel here")


# speedup vs baseline: 5.2592x; 5.2592x over previous
import functools

import jax
import jax.numpy as jnp
from jax import lax
from jax.experimental import pallas as pl
from jax.experimental.pallas import tpu as pltpu
from jax.experimental.pallas import tpu_sc as plsc

D = 64
CHUNK = 128
K = 8


@functools.lru_cache(maxsize=None)
def _make_gather(B: int):
    info = plsc.get_sparse_core_info()
    NC, NS = info.num_cores, info.num_subcores
    NW = NC * NS
    S = K * CHUNK
    assert B % (NW * S) == 0
    b_per_w = B // NW
    n_steps = b_per_w // S
    rows_per_step = S // CHUNK

    mesh = plsc.VectorSubcoreMesh(core_axis_name="c", subcore_axis_name="s")

    @functools.partial(
        pl.kernel,
        mesh=mesh,
        out_type=jax.ShapeDtypeStruct((B, D), jnp.float32),
        scratch_types=[
            pltpu.VMEM((K, CHUNK), jnp.int32),
            pltpu.VMEM((S, D), jnp.float32),
            pltpu.SemaphoreType.DMA,
        ],
        compiler_params=pltpu.CompilerParams(use_tc_tiling_on_sc=False),
    )
    def k(idx_hbm, table_hbm, out_hbm, idx_v, rows_v, sem):
        wid = lax.axis_index("s") * NC + lax.axis_index("c")
        base = wid * b_per_w

        def step(j, carry):
            start = base + j * S
            row0 = pl.multiple_of(start // CHUNK, 8)
            pltpu.sync_copy(idx_hbm.at[pl.ds(row0, rows_per_step)], idx_v)
            copies = []
            for t in range(K):
                copies.append(
                    pltpu.async_copy(
                        table_hbm.at[idx_v.at[t]],
                        rows_v.at[pl.ds(t * CHUNK, CHUNK)],
                        sem,
                    )
                )
            for c in copies:
                c.wait()
            pltpu.sync_copy(rows_v, out_hbm.at[pl.ds(start, S)])
            return carry

        lax.fori_loop(0, n_steps, step, 0)

    return k


def kernel(x, table):
    n, s = x.shape
    B = n * s
    idx2d = x.reshape(B // CHUNK, CHUNK).astype(jnp.int32)
    out = _make_gather(B)(idx2d, table.astype(jnp.float32))
    return out.reshape(n, s, D)
